# Initial kernel scaffold; baseline (speedup 1.0000x reference)
import functools

import jax
import jax.numpy as jnp
from jax import lax
from jax.experimental import pallas as pl
from jax.experimental.pallas import tpu as pltpu
from jax.experimental.pallas import tpu_sc as plsc

NC, NS, L = 2, 16, 16
IMAX = jnp.int32(2147483647)


def _h_body(x_ref, w1_ref, w2_ref, b2_ref, o_ref):
    xb = x_ref[...]
    a = jnp.dot(xb, w1_ref[...], preferred_element_type=jnp.float32)
    s = jnp.dot(xb, w2_ref[...], preferred_element_type=jnp.float32) + b2_ref[...]
    o_ref[...] = a * (1.0 / (1.0 + jnp.exp(-s)))


def _gated_h(x, W1, W2, b2):
    V, F = x.shape
    H = W1.shape[1]
    VB = 2000
    grid = (V // VB,)
    return pl.pallas_call(
        _h_body,
        grid=grid,
        in_specs=[
            pl.BlockSpec((VB, F), lambda i: (i, 0)),
            pl.BlockSpec((F, H), lambda i: (0, 0)),
            pl.BlockSpec((F, H), lambda i: (0, 0)),
            pl.BlockSpec((1, H), lambda i: (0, 0)),
        ],
        out_specs=pl.BlockSpec((VB, H), lambda i: (i, 0)),
        out_shape=jax.ShapeDtypeStruct((V, H), jnp.float32),
    )(x, W1, W2, b2.reshape(1, H))


def _out_body(z_ref, w3_ref, b3_ref, o_ref):
    o_ref[...] = (
        jnp.dot(z_ref[...], w3_ref[...], preferred_element_type=jnp.float32)
        + b3_ref[...]
    )


def _final_linear(z, W3, b3):
    G = z.shape[0]
    O = W3.shape[1]
    return pl.pallas_call(
        _out_body,
        out_shape=jax.ShapeDtypeStruct((G, O), jnp.float32),
    )(z, W3, b3.reshape(1, O))


def _make_segment_kernel(V, H, G, interpret=False):
    NW = NC * NS
    GPW = G // NW
    C = 256
    SPAD = 80
    mesh = plsc.VectorSubcoreMesh(
        core_axis_name="c", subcore_axis_name="s", num_cores=NC, num_subcores=NS
    )
    NK = H // L

    @functools.partial(
        pl.kernel,
        out_type=jax.ShapeDtypeStruct((G, 3 * H), jnp.float32),
        mesh=mesh,
        scratch_types=[
            pltpu.SMEM((SPAD,), jnp.int32),
            pltpu.VMEM((GPW, 3 * H), jnp.float32),
            pltpu.VMEM((C, H), jnp.float32),
        ],
        interpret=interpret,
    )
    def seg_kernel(h_hbm, n2g_hbm, out_hbm, starts_s, zout_v, chunk_v):
        wid = lax.axis_index("c") * NS + lax.axis_index("s")
        g0 = wid * GPW

        def phase1(ids_v):
            pltpu.sync_copy(n2g_hbm, ids_v)
            for k in range(SPAD // L):
                t = g0 + k * L + lax.iota(jnp.int32, (L,), 0)

                def bs_body(_, lohi):
                    lo, hi = lohi
                    mid = lax.shift_right_logical(lo + hi, 1)
                    val = plsc.load_gather(ids_v, [mid])
                    pred = val < t
                    return (
                        jnp.where(pred, mid + 1, lo),
                        jnp.where(pred, mid, hi),
                    )

                lo = jnp.zeros((L,), jnp.int32)
                hi = jnp.full((L,), V, jnp.int32)
                lo, hi = lax.fori_loop(0, 17, bs_body, (lo, hi))
                for l in range(L):
                    starts_s[k * L + l] = lo[l]

        pl.run_scoped(phase1, pltpu.VMEM((V,), jnp.int32))

        begin = starts_s[0]
        end = starts_s[GPW]

        def fresh():
            zs = tuple(jnp.zeros((L,), jnp.float32) for _ in range(NK))
            ms = tuple(jnp.full((L,), -jnp.inf, jnp.float32) for _ in range(NK))
            ags = tuple(jnp.full((L,), IMAX, jnp.int32) for _ in range(NK))
            return zs, ms, ags

        def flush(g, ss, ms, ags):
            gi = g - g0
            c = (starts_s[gi + 1] - starts_s[gi]).astype(jnp.float32)
            inv = 1.0 / jnp.maximum(c, 1.0)
            for k in range(NK):
                zout_v[gi, pl.ds(k * L, L)] = ss[k]
                zout_v[gi, pl.ds(H + k * L, L)] = ss[k] * inv
                zout_v[gi, pl.ds(2 * H + k * L, L)] = ags[k].astype(jnp.float32)

        def row_loop(carry):
            r, g, nb, cs, hi_row, ss, ms, ags = carry

            def adv_cond(st):
                return st[1] == r

            def adv_body(st):
                g_, nb_, ss_, ms_, ags_ = st
                flush(g_, ss_, ms_, ags_)
                g2 = g_ + 1
                nb2 = starts_s[g2 - g0 + 1]
                zs, zm, za = fresh()
                return (g2, nb2, zs, zm, za)

            g, nb, ss, ms, ags = lax.while_loop(
                adv_cond, adv_body, (g, nb, ss, ms, ags)
            )

            j = r - cs
            ss, ms, ags = list(ss), list(ms), list(ags)
            for k in range(NK):
                v = chunk_v[j, pl.ds(k * L, L)]
                ss[k] = ss[k] + v
                mflag = v > ms[k]
                ms[k] = jnp.where(mflag, v, ms[k])
                ags[k] = jnp.where(mflag, r, ags[k])
            return (r + 1, g, nb, cs, hi_row, tuple(ss), tuple(ms), tuple(ags))

        def chunk_body(carry):
            r, g, nb, ss, ms, ags = carry
            cs = jnp.minimum(r, V - C)
            pltpu.sync_copy(h_hbm.at[pl.ds(cs, C), :], chunk_v)
            hi_row = jnp.minimum(cs + C, end)
            st = lax.while_loop(
                lambda st: st[0] < st[4],
                row_loop,
                (r, g, nb, cs, hi_row, ss, ms, ags),
            )
            r, g, nb, _, _, ss, ms, ags = st
            return (r, g, nb, ss, ms, ags)

        zs0, zm0, za0 = fresh()
        nb0 = starts_s[1]
        st = lax.while_loop(
            lambda st: st[0] < end,
            chunk_body,
            (begin, g0, nb0, zs0, zm0, za0),
        )
        _, g, _, ss, ms, ags = st

        def tail_body(st):
            g_, ss_, ms_, ags_ = st
            flush(g_, ss_, ms_, ags_)
            zs, zm, za = fresh()
            return (g_ + 1, zs, zm, za)

        lax.while_loop(
            lambda st: st[0] < g0 + GPW, tail_body, (g, ss, ms, ags)
        )

        pltpu.sync_copy(zout_v, out_hbm.at[pl.ds(g0, GPW), :])

    return seg_kernel


def kernel(x, node2graph, W1, W2, b2, W3, b3):
    V, F = x.shape
    H = W1.shape[1]
    G = 2048
    h = _gated_h(x, W1, W2, b2)
    seg = _make_segment_kernel(V, H, G)
    z = seg(h, node2graph.astype(jnp.int32))
    return _final_linear(z, W3, b3)


# speedup vs baseline: 2.7018x; 2.7018x over previous
import functools

import jax
import jax.numpy as jnp
from jax import lax
from jax.experimental import pallas as pl
from jax.experimental.pallas import tpu as pltpu
from jax.experimental.pallas import tpu_sc as plsc

NC, NS, L = 2, 16, 16
IMAX = 2147483647


def _h_body(x_ref, w1_ref, w2_ref, b2_ref, o_ref):
    xb = x_ref[...]
    a = jnp.dot(xb, w1_ref[...], preferred_element_type=jnp.float32)
    s = jnp.dot(xb, w2_ref[...], preferred_element_type=jnp.float32) + b2_ref[...]
    o_ref[...] = a * (1.0 / (1.0 + jnp.exp(-s)))


def _gated_h(x, W1, W2, b2):
    V, F = x.shape
    H = W1.shape[1]
    VB = 2000
    grid = (V // VB,)
    return pl.pallas_call(
        _h_body,
        grid=grid,
        in_specs=[
            pl.BlockSpec((VB, F), lambda i: (i, 0)),
            pl.BlockSpec((F, H), lambda i: (0, 0)),
            pl.BlockSpec((F, H), lambda i: (0, 0)),
            pl.BlockSpec((1, H), lambda i: (0, 0)),
        ],
        out_specs=pl.BlockSpec((VB, H), lambda i: (i, 0)),
        out_shape=jax.ShapeDtypeStruct((V, H), jnp.float32),
    )(x, W1, W2, b2.reshape(1, H))


def _out_body(z_ref, w3_ref, b3_ref, o_ref):
    o_ref[...] = (
        jnp.dot(z_ref[...], w3_ref[...], preferred_element_type=jnp.float32)
        + b3_ref[...]
    )


def _final_linear(z, W3, b3):
    G = z.shape[0]
    O = W3.shape[1]
    return pl.pallas_call(
        _out_body,
        out_shape=jax.ShapeDtypeStruct((G, O), jnp.float32),
    )(z, W3, b3.reshape(1, O))


def _make_starts_kernel(V, G, interpret=False):
    NW = NC * NS
    GPW = G // NW
    SPAD = 80
    mesh = plsc.VectorSubcoreMesh(
        core_axis_name="c", subcore_axis_name="s", num_cores=NC, num_subcores=NS
    )

    @functools.partial(
        pl.kernel,
        out_type=jax.ShapeDtypeStruct((G + 2 * L, ), jnp.int32),
        mesh=mesh,
        scratch_types=[
            pltpu.VMEM((V,), jnp.int32),
            pltpu.VMEM((SPAD,), jnp.int32),
        ],
        compiler_params=pltpu.CompilerParams(needs_layout_passes=False),
        interpret=interpret,
    )
    def starts_kernel(n2g_hbm, out_hbm, ids_v, res_v):
        wid = lax.axis_index("c") * NS + lax.axis_index("s")
        g0 = wid * GPW
        pltpu.sync_copy(n2g_hbm, ids_v)
        for k in range(SPAD // L):
            t = g0 + k * L + lax.broadcasted_iota(jnp.int32, (L,), 0)

            lo = jnp.zeros((L,), jnp.int32)
            hi = jnp.full((L,), V, jnp.int32)
            for _ in range(17):
                mid = lax.shift_right_logical(lo + hi, 1)
                val = plsc.load_gather(ids_v, [mid])
                pred = val < t
                lo = jnp.where(pred, mid + 1, lo)
                hi = jnp.where(pred, mid, hi)
            res_v[pl.ds(k * L, L)] = lo
        pltpu.sync_copy(res_v, out_hbm.at[pl.ds(pl.multiple_of(g0, GPW), SPAD)])

    return starts_kernel


def _make_segment_kernel(V, H, G, interpret=False):
    NW = NC * NS
    GPW = G // NW
    C = 256
    SPAD = 80
    mesh = plsc.VectorSubcoreMesh(
        core_axis_name="c", subcore_axis_name="s", num_cores=NC, num_subcores=NS
    )
    NK = H // L

    @functools.partial(
        pl.kernel,
        out_type=jax.ShapeDtypeStruct((G, 3 * H), jnp.float32),
        mesh=mesh,
        scratch_types=[
            pltpu.SMEM((SPAD,), jnp.int32),
            pltpu.VMEM((SPAD,), jnp.int32),
            pltpu.VMEM((GPW, 3 * H), jnp.float32),
            pltpu.VMEM((C, H), jnp.float32),
        ],
        compiler_params=pltpu.CompilerParams(needs_layout_passes=False),
        interpret=interpret,
    )
    def seg_kernel(h_hbm, starts_hbm, out_hbm, starts_s, stv_v, zout_v, chunk_v):
        wid = lax.axis_index("c") * NS + lax.axis_index("s")
        g0 = wid * GPW

        pltpu.sync_copy(starts_hbm.at[pl.ds(pl.multiple_of(g0, GPW), SPAD)], stv_v)
        for k in range(SPAD // L):
            v = stv_v[pl.ds(k * L, L)]
            for l in range(L):
                starts_s[k * L + l] = v[l]

        begin = starts_s[0]
        end = starts_s[GPW]

        def fresh():
            zs = tuple(jnp.zeros((L,), jnp.float32) for _ in range(NK))
            ms = tuple(jnp.full((L,), -jnp.inf, jnp.float32) for _ in range(NK))
            ags = tuple(jnp.full((L,), IMAX, jnp.int32) for _ in range(NK))
            return zs, ms, ags

        def flush(g, ss, ms, ags):
            gi = g - g0
            c = jnp.full((L,), starts_s[gi + 1] - starts_s[gi], jnp.int32)
            inv = 1.0 / jnp.maximum(c.astype(jnp.float32), 1.0)
            for k in range(NK):
                zout_v[gi, pl.ds(k * L, L)] = ss[k]
                zout_v[gi, pl.ds(H + k * L, L)] = ss[k] * inv
                zout_v[gi, pl.ds(2 * H + k * L, L)] = ags[k].astype(jnp.float32)

        def row_loop(carry):
            r, g, nb, cs, hi_row, ss, ms, ags = carry

            def adv_cond(st):
                return st[1] == r

            def adv_body(st):
                g_, nb_, ss_, ms_, ags_ = st
                flush(g_, ss_, ms_, ags_)
                g2 = g_ + 1
                nb2 = starts_s[g2 - g0 + 1]
                zs, zm, za = fresh()
                return (g2, nb2, zs, zm, za)

            g, nb, ss, ms, ags = lax.while_loop(
                adv_cond, adv_body, (g, nb, ss, ms, ags)
            )

            j = r - cs
            ss, ms, ags = list(ss), list(ms), list(ags)
            for k in range(NK):
                v = chunk_v[j, pl.ds(k * L, L)]
                ss[k] = ss[k] + v
                mflag = v > ms[k]
                ms[k] = jnp.where(mflag, v, ms[k])
                ags[k] = jnp.where(mflag, r, ags[k])
            return (r + 1, g, nb, cs, hi_row, tuple(ss), tuple(ms), tuple(ags))

        def chunk_body(carry):
            r, g, nb, ss, ms, ags = carry
            cs = lax.shift_left(
                lax.shift_right_logical(jnp.minimum(r, V - C), 3), 3
            )
            cs = pl.multiple_of(cs, 8)
            pltpu.sync_copy(h_hbm.at[pl.ds(cs, C), :], chunk_v)
            hi_row = jnp.minimum(cs + C, end)
            st = lax.while_loop(
                lambda st: st[0] < st[4],
                row_loop,
                (r, g, nb, cs, hi_row, ss, ms, ags),
            )
            r, g, nb, _, _, ss, ms, ags = st
            return (r, g, nb, ss, ms, ags)

        zs0, zm0, za0 = fresh()
        nb0 = starts_s[1]
        st = lax.while_loop(
            lambda st: st[0] < end,
            chunk_body,
            (begin, g0, nb0, zs0, zm0, za0),
        )
        _, g, _, ss, ms, ags = st

        def tail_body(st):
            g_, ss_, ms_, ags_ = st
            flush(g_, ss_, ms_, ags_)
            zs, zm, za = fresh()
            return (g_ + 1, zs, zm, za)

        lax.while_loop(
            lambda st: st[0] < g0 + GPW, tail_body, (g, ss, ms, ags)
        )

        pltpu.sync_copy(zout_v, out_hbm.at[pl.ds(pl.multiple_of(g0, GPW), GPW), :])

    return seg_kernel


def kernel(x, node2graph, W1, W2, b2, W3, b3):
    V, F = x.shape
    H = W1.shape[1]
    G = 2048
    h = _gated_h(x, W1, W2, b2)
    starts = jnp.searchsorted(
        node2graph.astype(jnp.int32),
        jnp.arange(G + 2 * L, dtype=jnp.int32),
        side="left",
    ).astype(jnp.int32)
    z = _make_segment_kernel(V, H, G)(h, starts)
    return _final_linear(z, W3, b3)


# speedup vs baseline: 4.2491x; 1.5727x over previous
import functools

import jax
import jax.numpy as jnp
from jax import lax
from jax.experimental import pallas as pl
from jax.experimental.pallas import tpu as pltpu
from jax.experimental.pallas import tpu_sc as plsc

NC, NS, L = 2, 16, 16
IMAX = 2147483647


def _h_body(x_ref, w1_ref, w2_ref, b2_ref, o_ref):
    xb = x_ref[...]
    a = jnp.dot(xb, w1_ref[...], preferred_element_type=jnp.float32)
    s = jnp.dot(xb, w2_ref[...], preferred_element_type=jnp.float32) + b2_ref[...]
    o_ref[...] = a * (1.0 / (1.0 + jnp.exp(-s)))


def _gated_h(x, W1, W2, b2):
    V, F = x.shape
    H = W1.shape[1]
    VB = 2000
    grid = (V // VB,)
    return pl.pallas_call(
        _h_body,
        grid=grid,
        in_specs=[
            pl.BlockSpec((VB, F), lambda i: (i, 0)),
            pl.BlockSpec((F, H), lambda i: (0, 0)),
            pl.BlockSpec((F, H), lambda i: (0, 0)),
            pl.BlockSpec((1, H), lambda i: (0, 0)),
        ],
        out_specs=pl.BlockSpec((VB, H), lambda i: (i, 0)),
        out_shape=jax.ShapeDtypeStruct((V, H), jnp.float32),
    )(x, W1, W2, b2.reshape(1, H))


def _out_body(z_ref, w3_ref, b3_ref, o_ref):
    o_ref[...] = (
        jnp.dot(z_ref[...], w3_ref[...], preferred_element_type=jnp.float32)
        + b3_ref[...]
    )


def _final_linear(z, W3, b3):
    G = z.shape[0]
    O = W3.shape[1]
    return pl.pallas_call(
        _out_body,
        out_shape=jax.ShapeDtypeStruct((G, O), jnp.float32),
    )(z, W3, b3.reshape(1, O))


def _make_starts_kernel(V, G, interpret=False):
    NW = NC * NS
    GPW = G // NW
    SPAD = 80
    RW = 128
    NR = -(-V // RW)
    NIT = (NR - 1).bit_length()
    mesh = plsc.VectorSubcoreMesh(
        core_axis_name="c", subcore_axis_name="s", num_cores=NC, num_subcores=NS
    )

    @functools.partial(
        pl.kernel,
        out_type=jax.ShapeDtypeStruct((G + 2 * L,), jnp.int32),
        mesh=mesh,
        scratch_types=[
            pltpu.VMEM((NR, RW), jnp.int32),
            pltpu.VMEM((SPAD,), jnp.int32),
        ],
        compiler_params=pltpu.CompilerParams(needs_layout_passes=False),
        interpret=interpret,
    )
    def starts_kernel(n2g_hbm, out_hbm, ids_v, res_v):
        wid = lax.axis_index("c") * NS + lax.axis_index("s")
        g0 = wid * GPW
        pltpu.sync_copy(n2g_hbm, ids_v)
        iota = lax.broadcasted_iota(jnp.int32, (L,), 0)
        SL = RW - L

        for k in range(SPAD // L):

            def l_body(st2):
                l, acc = st2
                t = g0 + k * L + l
                tv = jnp.full((L,), t, jnp.int32)

                def bs_cond(bst):
                    return bst[0] < NIT

                def bs_body(bst):
                    i, lo, hi = bst
                    mid = lax.shift_right_logical(lo + hi, 1)
                    s = ids_v[mid, pl.ds(SL, L)]
                    nlt = jnp.sum((s < tv).astype(jnp.int32))
                    below = nlt == L
                    lo = jnp.where(below, mid + 1, lo)
                    hi = jnp.where(below, hi, mid)
                    return i + 1, lo, hi

                _, lo, _ = lax.while_loop(
                    bs_cond, bs_body,
                    (jnp.int32(0), jnp.int32(0), jnp.int32(NR)),
                )
                rowi = jnp.minimum(lo, NR - 1)
                cnt = jnp.int32(0)
                for p in range(RW // L):
                    seg = ids_v[rowi, pl.ds(p * L, L)]
                    cnt = cnt + jnp.sum((seg < tv).astype(jnp.int32))
                cnt = jnp.where(lo < NR, cnt, 0)
                acc = jnp.where(iota == l, lo * RW + cnt, acc)
                return l + 1, acc

            _, acc = lax.while_loop(
                lambda s2: s2[0] < L, l_body,
                (jnp.int32(0), jnp.zeros((L,), jnp.int32)),
            )
            res_v[pl.ds(k * L, L)] = acc
        pltpu.sync_copy(res_v, out_hbm.at[pl.ds(pl.multiple_of(g0, GPW), SPAD)])

    return starts_kernel


def _pad_n2g(n2g, V):
    RW = 128
    NR = -(-V // RW)
    pad = jnp.full((NR * RW - V,), IMAX, jnp.int32)
    return jnp.concatenate([n2g.astype(jnp.int32), pad]).reshape(NR, RW)


def _make_segment_kernel(V, H, G, interpret=False):
    NW = NC * NS
    GPW = G // NW
    C = 256
    SPAD = 80
    mesh = plsc.VectorSubcoreMesh(
        core_axis_name="c", subcore_axis_name="s", num_cores=NC, num_subcores=NS
    )
    NK = H // L

    @functools.partial(
        pl.kernel,
        out_type=jax.ShapeDtypeStruct((G, 3 * H), jnp.float32),
        mesh=mesh,
        scratch_types=[
            pltpu.SMEM((SPAD,), jnp.int32),
            pltpu.VMEM((SPAD,), jnp.int32),
            pltpu.VMEM((GPW, 3 * H), jnp.float32),
            pltpu.VMEM((C, H), jnp.float32),
        ],
        compiler_params=pltpu.CompilerParams(needs_layout_passes=False),
        interpret=interpret,
    )
    def seg_kernel(h_hbm, starts_hbm, out_hbm, starts_s, stv_v, zout_v, chunk_v):
        wid = lax.axis_index("c") * NS + lax.axis_index("s")
        g0 = wid * GPW

        pltpu.sync_copy(starts_hbm.at[pl.ds(pl.multiple_of(g0, GPW), SPAD)], stv_v)
        for k in range(SPAD // L):
            v = stv_v[pl.ds(k * L, L)]
            for l in range(L):
                starts_s[k * L + l] = v[l]

        begin = starts_s[0]
        end = starts_s[GPW]

        def fresh():
            zs = tuple(jnp.zeros((L,), jnp.float32) for _ in range(NK))
            ms = tuple(jnp.full((L,), -jnp.inf, jnp.float32) for _ in range(NK))
            ags = tuple(jnp.full((L,), IMAX, jnp.int32) for _ in range(NK))
            return zs, ms, ags

        def flush(g, ss, ms, ags):
            gi = g - g0
            c = jnp.full((L,), starts_s[gi + 1] - starts_s[gi], jnp.int32)
            inv = 1.0 / jnp.maximum(c.astype(jnp.float32), 1.0)
            for k in range(NK):
                zout_v[gi, pl.ds(k * L, L)] = ss[k]
                zout_v[gi, pl.ds(H + k * L, L)] = ss[k] * inv
                zout_v[gi, pl.ds(2 * H + k * L, L)] = ags[k].astype(jnp.float32)

        def row_loop(carry):
            r, g, nb, cs, hi_row, ss, ms, ags = carry

            def adv_cond(st):
                return st[1] == r

            def adv_body(st):
                g_, nb_, ss_, ms_, ags_ = st
                flush(g_, ss_, ms_, ags_)
                g2 = g_ + 1
                nb2 = starts_s[g2 - g0 + 1]
                zs, zm, za = fresh()
                return (g2, nb2, zs, zm, za)

            g, nb, ss, ms, ags = lax.while_loop(
                adv_cond, adv_body, (g, nb, ss, ms, ags)
            )

            j = r - cs
            ss, ms, ags = list(ss), list(ms), list(ags)
            for k in range(NK):
                v = chunk_v[j, pl.ds(k * L, L)]
                ss[k] = ss[k] + v
                mflag = v > ms[k]
                ms[k] = jnp.where(mflag, v, ms[k])
                ags[k] = jnp.where(mflag, r, ags[k])
            return (r + 1, g, nb, cs, hi_row, tuple(ss), tuple(ms), tuple(ags))

        def chunk_body(carry):
            r, g, nb, ss, ms, ags = carry
            cs = lax.shift_left(
                lax.shift_right_logical(jnp.minimum(r, V - C), 3), 3
            )
            cs = pl.multiple_of(cs, 8)
            pltpu.sync_copy(h_hbm.at[pl.ds(cs, C), :], chunk_v)
            hi_row = jnp.minimum(cs + C, end)
            st = lax.while_loop(
                lambda st: st[0] < st[4],
                row_loop,
                (r, g, nb, cs, hi_row, ss, ms, ags),
            )
            r, g, nb, _, _, ss, ms, ags = st
            return (r, g, nb, ss, ms, ags)

        zs0, zm0, za0 = fresh()
        nb0 = starts_s[1]
        st = lax.while_loop(
            lambda st: st[0] < end,
            chunk_body,
            (begin, g0, nb0, zs0, zm0, za0),
        )
        _, g, _, ss, ms, ags = st

        def tail_body(st):
            g_, ss_, ms_, ags_ = st
            flush(g_, ss_, ms_, ags_)
            zs, zm, za = fresh()
            return (g_ + 1, zs, zm, za)

        lax.while_loop(
            lambda st: st[0] < g0 + GPW, tail_body, (g, ss, ms, ags)
        )

        pltpu.sync_copy(zout_v, out_hbm.at[pl.ds(pl.multiple_of(g0, GPW), GPW), :])

    return seg_kernel


def kernel(x, node2graph, W1, W2, b2, W3, b3):
    V, F = x.shape
    H = W1.shape[1]
    G = 2048
    h = _gated_h(x, W1, W2, b2)
    starts = _make_starts_kernel(V, G)(_pad_n2g(node2graph, V))
    z = _make_segment_kernel(V, H, G)(h, starts)
    return _final_linear(z, W3, b3)


# speedup vs baseline: 4.3044x; 1.0130x over previous
import functools

import jax
import jax.numpy as jnp
from jax import lax
from jax.experimental import pallas as pl
from jax.experimental.pallas import tpu as pltpu
from jax.experimental.pallas import tpu_sc as plsc

NC, NS, L = 2, 16, 16
IMAX = 2147483647


def _h_body(x_ref, w1_ref, w2_ref, b2_ref, o_ref):
    xb = x_ref[...]
    a = jnp.dot(xb, w1_ref[...], preferred_element_type=jnp.float32)
    s = jnp.dot(xb, w2_ref[...], preferred_element_type=jnp.float32) + b2_ref[...]
    o_ref[...] = a * (1.0 / (1.0 + jnp.exp(-s)))


def _gated_h(x, W1, W2, b2):
    V, F = x.shape
    H = W1.shape[1]
    VB = 2000
    grid = (V // VB,)
    return pl.pallas_call(
        _h_body,
        grid=grid,
        in_specs=[
            pl.BlockSpec((VB, F), lambda i: (i, 0)),
            pl.BlockSpec((F, H), lambda i: (0, 0)),
            pl.BlockSpec((F, H), lambda i: (0, 0)),
            pl.BlockSpec((1, H), lambda i: (0, 0)),
        ],
        out_specs=pl.BlockSpec((VB, H), lambda i: (i, 0)),
        out_shape=jax.ShapeDtypeStruct((V, H), jnp.float32),
    )(x, W1, W2, b2.reshape(1, H))


def _out_body(z_ref, w3_ref, b3_ref, o_ref):
    o_ref[...] = (
        jnp.dot(z_ref[...], w3_ref[...], preferred_element_type=jnp.float32)
        + b3_ref[...]
    )


def _final_linear(z, W3, b3):
    G = z.shape[0]
    O = W3.shape[1]
    return pl.pallas_call(
        _out_body,
        out_shape=jax.ShapeDtypeStruct((G, O), jnp.float32),
    )(z, W3, b3.reshape(1, O))


def _make_starts_kernel(V, G, interpret=False):
    NW = NC * NS
    GPW = G // NW
    SPAD = 80
    RW = 128
    NR = -(-V // RW)
    NIT = (NR - 1).bit_length()
    mesh = plsc.VectorSubcoreMesh(
        core_axis_name="c", subcore_axis_name="s", num_cores=NC, num_subcores=NS
    )

    @functools.partial(
        pl.kernel,
        out_type=jax.ShapeDtypeStruct((G + 2 * L,), jnp.int32),
        mesh=mesh,
        scratch_types=[
            pltpu.VMEM((NR, RW), jnp.int32),
            pltpu.VMEM((SPAD,), jnp.int32),
        ],
        compiler_params=pltpu.CompilerParams(needs_layout_passes=False),
        interpret=interpret,
    )
    def starts_kernel(n2g_hbm, out_hbm, ids_v, res_v):
        wid = lax.axis_index("c") * NS + lax.axis_index("s")
        g0 = wid * GPW
        pltpu.sync_copy(n2g_hbm, ids_v)
        iota = lax.broadcasted_iota(jnp.int32, (L,), 0)
        SL = RW - L

        for k in range(SPAD // L):

            def l_body(st2):
                l, acc = st2
                t = g0 + k * L + l
                tv = jnp.full((L,), t, jnp.int32)

                def bs_cond(bst):
                    return bst[0] < NIT

                def bs_body(bst):
                    i, lo, hi = bst
                    mid = lax.shift_right_logical(lo + hi, 1)
                    s = ids_v[mid, pl.ds(SL, L)]
                    nlt = jnp.sum((s < tv).astype(jnp.int32))
                    below = nlt == L
                    lo = jnp.where(below, mid + 1, lo)
                    hi = jnp.where(below, hi, mid)
                    return i + 1, lo, hi

                _, lo, _ = lax.while_loop(
                    bs_cond, bs_body,
                    (jnp.int32(0), jnp.int32(0), jnp.int32(NR)),
                )
                rowi = jnp.minimum(lo, NR - 1)
                cnt = jnp.int32(0)
                for p in range(RW // L):
                    seg = ids_v[rowi, pl.ds(p * L, L)]
                    cnt = cnt + jnp.sum((seg < tv).astype(jnp.int32))
                cnt = jnp.where(lo < NR, cnt, 0)
                acc = jnp.where(iota == l, lo * RW + cnt, acc)
                return l + 1, acc

            _, acc = lax.while_loop(
                lambda s2: s2[0] < L, l_body,
                (jnp.int32(0), jnp.zeros((L,), jnp.int32)),
            )
            res_v[pl.ds(k * L, L)] = acc
        pltpu.sync_copy(res_v, out_hbm.at[pl.ds(pl.multiple_of(g0, GPW), SPAD)])

    return starts_kernel


def _pad_n2g(n2g, V):
    RW = 128
    NR = -(-V // RW)
    pad = jnp.full((NR * RW - V,), IMAX, jnp.int32)
    return jnp.concatenate([n2g.astype(jnp.int32), pad]).reshape(NR, RW)


def _make_segment_kernel(V, H, G, interpret=False):
    NW = NC * NS
    GPW = G // NW
    C = 512
    SPAD = 80
    mesh = plsc.VectorSubcoreMesh(
        core_axis_name="c", subcore_axis_name="s", num_cores=NC, num_subcores=NS
    )
    NK = H // L

    @functools.partial(
        pl.kernel,
        out_type=jax.ShapeDtypeStruct((G, 3 * H), jnp.float32),
        mesh=mesh,
        scratch_types=[
            pltpu.SMEM((SPAD,), jnp.int32),
            pltpu.VMEM((SPAD,), jnp.int32),
            pltpu.VMEM((GPW, 3 * H), jnp.float32),
            pltpu.VMEM((C, H), jnp.float32),
        ],
        compiler_params=pltpu.CompilerParams(needs_layout_passes=False),
        interpret=interpret,
    )
    def seg_kernel(h_hbm, starts_hbm, out_hbm, starts_s, stv_v, zout_v, chunk_v):
        wid = lax.axis_index("c") * NS + lax.axis_index("s")
        g0 = wid * GPW

        pltpu.sync_copy(starts_hbm.at[pl.ds(pl.multiple_of(g0, GPW), SPAD)], stv_v)
        for k in range(SPAD // L):
            v = stv_v[pl.ds(k * L, L)]
            for l in range(L):
                starts_s[k * L + l] = v[l]

        begin = starts_s[0]
        end = starts_s[GPW]

        def fresh():
            zs = tuple(jnp.zeros((L,), jnp.float32) for _ in range(NK))
            ms = tuple(jnp.full((L,), -jnp.inf, jnp.float32) for _ in range(NK))
            ags = tuple(jnp.full((L,), IMAX, jnp.int32) for _ in range(NK))
            return zs, ms, ags

        def flush(g, ss, ms, ags):
            gi = g - g0
            c = jnp.full((L,), starts_s[gi + 1] - starts_s[gi], jnp.int32)
            inv = 1.0 / jnp.maximum(c.astype(jnp.float32), 1.0)
            for k in range(NK):
                zout_v[gi, pl.ds(k * L, L)] = ss[k]
                zout_v[gi, pl.ds(H + k * L, L)] = ss[k] * inv
                zout_v[gi, pl.ds(2 * H + k * L, L)] = ags[k].astype(jnp.float32)

        def row_loop(carry):
            r, g, nb, cs, hi_row, ss, ms, ags = carry

            def adv_cond(st):
                return st[1] == r

            def adv_body(st):
                g_, nb_, ss_, ms_, ags_ = st
                flush(g_, ss_, ms_, ags_)
                g2 = g_ + 1
                nb2 = starts_s[g2 - g0 + 1]
                zs, zm, za = fresh()
                return (g2, nb2, zs, zm, za)

            g, nb, ss, ms, ags = lax.while_loop(
                adv_cond, adv_body, (g, nb, ss, ms, ags)
            )

            j = r - cs
            ss, ms, ags = list(ss), list(ms), list(ags)
            for k in range(NK):
                v = chunk_v[j, pl.ds(k * L, L)]
                ss[k] = ss[k] + v
                mflag = v > ms[k]
                ms[k] = jnp.where(mflag, v, ms[k])
                ags[k] = jnp.where(mflag, r, ags[k])
            return (r + 1, g, nb, cs, hi_row, tuple(ss), tuple(ms), tuple(ags))

        def chunk_body(carry):
            r, g, nb, ss, ms, ags = carry
            cs = lax.shift_left(
                lax.shift_right_logical(jnp.minimum(r, V - C), 3), 3
            )
            cs = pl.multiple_of(cs, 8)
            pltpu.sync_copy(h_hbm.at[pl.ds(cs, C), :], chunk_v)
            hi_row = jnp.minimum(cs + C, end)
            st = lax.while_loop(
                lambda st: st[0] < st[4],
                row_loop,
                (r, g, nb, cs, hi_row, ss, ms, ags),
            )
            r, g, nb, _, _, ss, ms, ags = st
            return (r, g, nb, ss, ms, ags)

        zs0, zm0, za0 = fresh()
        nb0 = starts_s[1]
        st = lax.while_loop(
            lambda st: st[0] < end,
            chunk_body,
            (begin, g0, nb0, zs0, zm0, za0),
        )
        _, g, _, ss, ms, ags = st

        def tail_body(st):
            g_, ss_, ms_, ags_ = st
            flush(g_, ss_, ms_, ags_)
            zs, zm, za = fresh()
            return (g_ + 1, zs, zm, za)

        lax.while_loop(
            lambda st: st[0] < g0 + GPW, tail_body, (g, ss, ms, ags)
        )

        pltpu.sync_copy(zout_v, out_hbm.at[pl.ds(pl.multiple_of(g0, GPW), GPW), :])

    return seg_kernel


def kernel(x, node2graph, W1, W2, b2, W3, b3):
    V, F = x.shape
    H = W1.shape[1]
    G = 2048
    h = _gated_h(x, W1, W2, b2)
    starts = _make_starts_kernel(V, G)(_pad_n2g(node2graph, V))
    z = _make_segment_kernel(V, H, G)(h, starts)
    return _final_linear(z, W3, b3)
